# baseline (speedup 1.0000x reference)
import functools

import jax
import jax.numpy as jnp
from jax import lax
from jax.experimental import pallas as pl
from jax.experimental.pallas import tpu as pltpu
from jax.experimental.pallas import tpu_sc as plsc

N = 10000
E = 160000
NSUB = 16
EDGE_B = 96
NB = 108
EP = NSUB * NB * EDGE_B
NBT = 2 * NB


def _matmul_proj(x, wl, b, wc_r, bn):
    n, k = x.shape
    dout = wl.shape[1]
    half = wc_r.shape[2]

    def mm(x_ref, wl_ref, b_ref, wc_ref, lin_ref, m_ref):
        xv = x_ref[...]
        lin_ref[...] = jnp.dot(xv, wl_ref[...],
                               preferred_element_type=jnp.float32) + b_ref[...]
        m_ref[0] = jnp.dot(xv, wc_ref[0],
                           preferred_element_type=jnp.float32)

    return pl.pallas_call(
        mm,
        grid=(n // bn, 4),
        in_specs=[
            pl.BlockSpec((bn, k), lambda i, j: (i, 0)),
            pl.BlockSpec((k, dout), lambda i, j: (0, 0)),
            pl.BlockSpec((1, dout), lambda i, j: (0, 0)),
            pl.BlockSpec((1, k, half), lambda i, j: (j, 0, 0)),
        ],
        out_specs=[
            pl.BlockSpec((bn, dout), lambda i, j: (i, 0)),
            pl.BlockSpec((1, bn, half),
                         lambda i, j: (j % 2, (j // 2) * (n // bn) + i, 0)),
        ],
        out_shape=[
            jax.ShapeDtypeStruct((n, dout), jnp.float32),
            jax.ShapeDtypeStruct((2, 2 * n, half), jnp.float32),
        ],
    )(x, wl, b[None, :], wc_r)


def _log_softmax(h, bn):
    n, c = h.shape

    def k(h_ref, o_ref):
        v = h_ref[...]
        mx = jnp.max(v, axis=1, keepdims=True)
        e = jnp.exp(v - mx)
        o_ref[...] = v - mx - jnp.log(jnp.sum(e, axis=1, keepdims=True))

    return pl.pallas_call(
        k,
        grid=(n // bn,),
        in_specs=[pl.BlockSpec((bn, c), lambda i: (i, 0))],
        out_specs=pl.BlockSpec((bn, c), lambda i: (i, 0)),
        out_shape=jax.ShapeDtypeStruct((n, c), jnp.float32),
    )(h)


@functools.partial(jax.jit, static_argnames=("half",))
def _sc_aggregate(lin, m, ed, ew, half):
    rows_per_tile = 632
    mesh = plsc.VectorSubcoreMesh(core_axis_name="c", subcore_axis_name="s")

    @functools.partial(
        pl.kernel,
        mesh=mesh,
        compiler_params=pltpu.CompilerParams(use_tc_tiling_on_sc=False,
                                             needs_layout_passes=False),
        out_type=jax.ShapeDtypeStruct((N, 2 * half), jnp.float32),
        scratch_types=[
            pltpu.VMEM((4, 2, EDGE_B), jnp.int32),
            pltpu.VMEM((4, EDGE_B), jnp.float32),
            pltpu.VMEM((4, EDGE_B), jnp.int32),
            pltpu.VMEM((EDGE_B, half), jnp.float32),
            pltpu.VMEM((EDGE_B, half), jnp.float32),
            pltpu.VMEM((EDGE_B, half), jnp.float32),
            pltpu.VMEM((EDGE_B, half), jnp.float32),
            pltpu.VMEM_SHARED((N, half), jnp.float32),
            pltpu.SemaphoreType.DMA,
            pltpu.SemaphoreType.DMA,
            pltpu.SemaphoreType.DMA,
            pltpu.SemaphoreType.DMA,
            pltpu.SemaphoreType.DMA,
            pltpu.SemaphoreType.DMA,
            pltpu.SemaphoreType.DMA,
            pltpu.SemaphoreType.DMA,
            pltpu.SemaphoreType.DMA,
            pltpu.SemaphoreType.DMA,
            pltpu.SemaphoreType.DMA,
            pltpu.SemaphoreType.DMA,
        ],
    )
    def agg(lin_h, m_h, ed_h, ew_h, out_h,
            ib, wb, sd, r0b, r1b, r2b, r3b, acc,
            si0, si1, si2, si3, sg0, sg1, sg2, sg3, ss0, ss1, ss2, ss3):
        cid = lax.axis_index("c")
        sid = lax.axis_index("s")
        r0 = pl.multiple_of(
            jnp.minimum(sid * rows_per_tile, N - rows_per_tile), 8)
        rows = (r0b, r1b, r2b, r3b)
        sem_i = (si0, si1, si2, si3)
        sem_g = (sg0, sg1, sg2, sg3)
        sem_s = (ss0, ss1, ss2, ss3)

        def one_core(coff, mc_h):
            pltpu.sync_copy(lin_h.at[pl.ds(r0, rows_per_tile),
                                     pl.ds(coff, half)],
                            acc.at[pl.ds(r0, rows_per_tile)])
            plsc.subcore_barrier()

            def load_idx(b, p):
                pltpu.async_copy(ed_h.at[sid, b], ib.at[p], sem_i[p])
                pltpu.async_copy(ew_h.at[sid, b], wb.at[p], sem_i[p])

            def wait_idx(p):
                pltpu.make_async_copy(ed_h.at[sid, 0], ib.at[p],
                                      sem_i[p]).wait()
                pltpu.make_async_copy(ew_h.at[sid, 0], wb.at[p],
                                      sem_i[p]).wait()

            def start_gather(p):
                pltpu.async_copy(mc_h.at[ib.at[p, 0]], rows[p], sem_g[p])

            def wait_gather(p):
                pltpu.make_async_copy(mc_h.at[ib.at[p, 0]],
                                      rows[p], sem_g[p]).wait()

            def start_scatter(p):
                pltpu.async_copy(rows[p], acc.at[sd.at[p]],
                                 sem_s[p], add=True)

            def wait_scatter(p):
                pltpu.make_async_copy(rows[p], acc.at[sd.at[p]],
                                      sem_s[p]).wait()

            def scale(p):
                rbuf = rows[p]

                def scale_group(g, c2):
                    goff = pl.multiple_of(g * 16, 16)
                    sl = pl.ds(goff, 16)
                    sd[p, sl] = ib[p, 1, sl]
                    w16 = wb[p, sl]
                    for t in range(16):
                        wbc = w16.at[jnp.full((16,), t, jnp.int32)].get(
                            mode="promise_in_bounds")
                        for q in range(half // 16):
                            qsl = pl.ds(q * 16, 16)
                            rbuf[goff + t, qsl] = rbuf[goff + t, qsl] * wbc
                    return c2

                lax.fori_loop(0, EDGE_B // 16, scale_group, 0)

            for p in range(4):
                load_idx(p, p)
            wait_idx(0)
            start_gather(0)
            wait_idx(1)
            start_gather(1)

            def step(b, p):
                wait_gather(p)

                @pl.when(b >= 2)
                def _():
                    wait_scatter((p + 2) % 4)

                @pl.when(b + 2 < NBT)
                def _():
                    wait_idx((p + 2) % 4)
                    start_gather((p + 2) % 4)

                scale(p)
                start_scatter(p)

                @pl.when(b + 4 < NBT)
                def _():
                    load_idx(b + 4, p)

            def quad(k4, c):
                for i in range(4):
                    step(4 * k4 + i, i)
                return c

            lax.fori_loop(0, NBT // 4, quad, 0)
            wait_scatter((NBT - 2) % 4)
            wait_scatter((NBT - 1) % 4)

            plsc.subcore_barrier()
            pltpu.sync_copy(acc.at[pl.ds(r0, rows_per_tile)],
                            out_h.at[pl.ds(r0, rows_per_tile),
                                     pl.ds(coff, half)])

        @pl.when(cid == 0)
        def _():
            one_core(0, m_h.at[0])

        @pl.when(cid == 1)
        def _():
            one_core(half, m_h.at[1])

    return agg(lin, m, ed, ew)


def _pack_edges(edge_index, edge_weight, edge_index2, edge_weight2):
    pad = EP - E

    def one(ei, ew, src_off):
        src = jnp.concatenate([ei[0] + src_off,
                               jnp.full((pad,), src_off, jnp.int32)])
        dst = jnp.concatenate([ei[1], jnp.zeros((pad,), jnp.int32)])
        ewp = jnp.concatenate([ew, jnp.zeros((pad,), jnp.float32)])
        packed = jnp.stack([src, dst], axis=0).reshape(2, NSUB, NB, EDGE_B)
        return (jnp.transpose(packed, (1, 2, 0, 3)),
                ewp.reshape(NSUB, NB, EDGE_B))

    ed1, ew1 = one(edge_index, edge_weight, 0)
    ed2, ew2 = one(edge_index2, edge_weight2, N)
    return (jnp.concatenate([ed1, ed2], axis=1),
            jnp.concatenate([ew1, ew2], axis=1))


def _layer(h, wl, wc1, wc2, bl, bc1, bc2, ed, ew):
    dout = wl.shape[1]
    half = dout // 2
    wc_r = jnp.stack([wc1[:, :half], wc1[:, half:],
                      wc2[:, :half], wc2[:, half:]], axis=0)
    lin, m = _matmul_proj(h, wl, bl + bc1 + bc2, wc_r, bn=1000)
    return _sc_aggregate(lin, m, ed, ew, half=half)


def kernel(x, edge_index, edge_weight, edge_index2, edge_weight2,
           Wl1, Wc11, Wc21, bl1, bc11, bc21,
           Wl2, Wc12, Wc22, bl2, bc12, bc22,
           Wl3, Wc13, Wc23, bl3, bc13, bc23):
    ed, ew = _pack_edges(edge_index, edge_weight, edge_index2, edge_weight2)

    h = _layer(x, Wl1, Wc11, Wc21, bl1, bc11, bc21, ed, ew)
    h = _layer(h, Wl2, Wc12, Wc22, bl2, bc12, bc22, ed, ew)
    h = _layer(h, Wl3, Wc13, Wc23, bl3, bc13, bc23, ed, ew)
    return _log_softmax(h, bn=1000)


# speedup vs baseline: 1.5974x; 1.5974x over previous
import functools

import jax
import jax.numpy as jnp
from jax import lax
from jax.experimental import pallas as pl
from jax.experimental.pallas import tpu as pltpu
from jax.experimental.pallas import tpu_sc as plsc

N = 10000
E = 160000
NSUB = 16
EDGE_B = 112
NB = 90
EP = NSUB * NB * EDGE_B


def _matmul_bias(x, w, b, bn):
    n, k = x.shape
    m = w.shape[1]

    def mm(x_ref, w_ref, b_ref, o_ref):
        o_ref[...] = jnp.dot(x_ref[...], w_ref[...],
                             preferred_element_type=jnp.float32) + b_ref[...]

    return pl.pallas_call(
        mm,
        grid=(n // bn,),
        in_specs=[
            pl.BlockSpec((bn, k), lambda i: (i, 0)),
            pl.BlockSpec((k, m), lambda i: (0, 0)),
            pl.BlockSpec((1, m), lambda i: (0, 0)),
        ],
        out_specs=pl.BlockSpec((bn, m), lambda i: (i, 0)),
        out_shape=jax.ShapeDtypeStruct((n, m), jnp.float32),
    )(x, w, b[None, :])


def _log_softmax(h, bn):
    n, c = h.shape

    def k(h_ref, o_ref):
        v = h_ref[...]
        mx = jnp.max(v, axis=1, keepdims=True)
        e = jnp.exp(v - mx)
        o_ref[...] = v - mx - jnp.log(jnp.sum(e, axis=1, keepdims=True))

    return pl.pallas_call(
        k,
        grid=(n // bn,),
        in_specs=[pl.BlockSpec((bn, c), lambda i: (i, 0))],
        out_specs=pl.BlockSpec((bn, c), lambda i: (i, 0)),
        out_shape=jax.ShapeDtypeStruct((n, c), jnp.float32),
    )(h)


@functools.partial(jax.jit, static_argnames=("half",))
def _sc_aggregate(lin_a, lin_b, m1_a, m1_b, m2_a, m2_b,
                  ed1, ew1, ed2, ew2, half):
    rows_per_tile = 632
    mesh = plsc.VectorSubcoreMesh(core_axis_name="c", subcore_axis_name="s")

    @functools.partial(
        pl.kernel,
        mesh=mesh,
        compiler_params=pltpu.CompilerParams(use_tc_tiling_on_sc=False),
        out_type=(jax.ShapeDtypeStruct((N, half), jnp.float32),
                  jax.ShapeDtypeStruct((N, half), jnp.float32)),
        scratch_types=[
            pltpu.VMEM((3, 2, EDGE_B), jnp.int32),
            pltpu.VMEM((3, EDGE_B), jnp.float32),
            pltpu.VMEM((3, EDGE_B), jnp.int32),
            pltpu.VMEM((EDGE_B, half), jnp.float32),
            pltpu.VMEM((EDGE_B, half), jnp.float32),
            pltpu.VMEM((EDGE_B, half), jnp.float32),
            pltpu.VMEM_SHARED((N, half), jnp.float32),
            pltpu.SemaphoreType.DMA,
            pltpu.SemaphoreType.DMA,
            pltpu.SemaphoreType.DMA,
            pltpu.SemaphoreType.DMA,
            pltpu.SemaphoreType.DMA,
            pltpu.SemaphoreType.DMA,
            pltpu.SemaphoreType.DMA,
            pltpu.SemaphoreType.DMA,
            pltpu.SemaphoreType.DMA,
        ],
    )
    def agg(lin_a_h, lin_b_h, m1_a_h, m1_b_h, m2_a_h, m2_b_h,
            ed1_h, ew1_h, ed2_h, ew2_h, out_a_h, out_b_h,
            ib, wb, sd, rows0, rows1, rows2, acc,
            si0, si1, si2, sg0, sg1, sg2, ss0, ss1, ss2):
        cid = lax.axis_index("c")
        sid = lax.axis_index("s")
        r0 = pl.multiple_of(
            jnp.minimum(sid * rows_per_tile, N - rows_per_tile), 8)
        rows = (rows0, rows1, rows2)
        sem_i = (si0, si1, si2)
        sem_g = (sg0, sg1, sg2)
        sem_s = (ss0, ss1, ss2)

        def one_core(lin_h, m1_h, m2_h, out_h):
            pltpu.sync_copy(lin_h.at[pl.ds(r0, rows_per_tile)],
                            acc.at[pl.ds(r0, rows_per_tile)])
            plsc.subcore_barrier()

            def one_conv(m_h, ed_h, ew_h):
                def load_idx(b, p):
                    pltpu.async_copy(ed_h.at[sid, b], ib.at[p], sem_i[p])
                    pltpu.async_copy(ew_h.at[sid, b], wb.at[p], sem_i[p])

                def wait_idx(p):
                    pltpu.make_async_copy(ed_h.at[sid, 0], ib.at[p],
                                          sem_i[p]).wait()
                    pltpu.make_async_copy(ew_h.at[sid, 0], wb.at[p],
                                          sem_i[p]).wait()

                def start_gather(p):
                    pltpu.async_copy(m_h.at[ib.at[p, 0]], rows[p], sem_g[p])

                def wait_gather(p):
                    pltpu.make_async_copy(m_h.at[ib.at[p, 0]],
                                          rows[p], sem_g[p]).wait()

                def start_scatter(p):
                    pltpu.async_copy(rows[p], acc.at[sd.at[p]],
                                     sem_s[p], add=True)

                def wait_scatter(p):
                    pltpu.make_async_copy(rows[p], acc.at[sd.at[p]],
                                          sem_s[p]).wait()

                def scale(p):
                    rbuf = rows[p]

                    def scale_group(g, c2):
                        goff = pl.multiple_of(g * 16, 16)
                        sl = pl.ds(goff, 16)
                        sd[p, sl] = ib[p, 1, sl]
                        w16 = wb[p, sl]
                        for t in range(16):
                            wbc = w16.at[jnp.full((16,), t, jnp.int32)].get(
                                mode="promise_in_bounds")
                            for q in range(half // 16):
                                qsl = pl.ds(q * 16, 16)
                                rbuf[goff + t, qsl] = rbuf[goff + t, qsl] * wbc
                        return c2

                    lax.fori_loop(0, EDGE_B // 16, scale_group, 0)

                for p in range(3):
                    load_idx(p, p)
                wait_idx(0)
                start_gather(0)

                def step(b, p):
                    wait_gather(p)

                    @pl.when(b >= 2)
                    def _():
                        wait_scatter((p + 1) % 3)

                    @pl.when(b + 1 < NB)
                    def _():
                        wait_idx((p + 1) % 3)
                        start_gather((p + 1) % 3)

                    scale(p)
                    start_scatter(p)

                    @pl.when(b + 3 < NB)
                    def _():
                        load_idx(b + 3, p)

                def trio(k3, c):
                    for i in range(3):
                        step(3 * k3 + i, i)
                    return c

                lax.fori_loop(0, NB // 3, trio, 0)
                wait_scatter((NB - 2) % 3)
                wait_scatter((NB - 1) % 3)

            one_conv(m1_h, ed1_h, ew1_h)
            one_conv(m2_h, ed2_h, ew2_h)
            plsc.subcore_barrier()
            pltpu.sync_copy(acc.at[pl.ds(r0, rows_per_tile)],
                            out_h.at[pl.ds(r0, rows_per_tile)])

        @pl.when(cid == 0)
        def _():
            one_core(lin_a_h, m1_a_h, m2_a_h, out_a_h)

        @pl.when(cid == 1)
        def _():
            one_core(lin_b_h, m1_b_h, m2_b_h, out_b_h)

    return agg(lin_a, lin_b, m1_a, m1_b, m2_a, m2_b, ed1, ew1, ed2, ew2)


def _pack_edges(edge_index, edge_weight):
    pad = EP - E
    src = jnp.concatenate([edge_index[0], jnp.zeros((pad,), jnp.int32)])
    dst = jnp.concatenate([edge_index[1], jnp.zeros((pad,), jnp.int32)])
    ewp = jnp.concatenate([edge_weight, jnp.zeros((pad,), jnp.float32)])
    packed = jnp.stack([src, dst], axis=0).reshape(2, NSUB, NB, EDGE_B)
    return (jnp.transpose(packed, (1, 2, 0, 3)),
            ewp.reshape(NSUB, NB, EDGE_B))


def _layer(h, wl, wc1, wc2, bl, bc1, bc2, edges1, edges2):
    dout = wl.shape[1]
    half = dout // 2
    wcat = jnp.concatenate([wl, wc1, wc2], axis=1)
    bcat = jnp.concatenate([bl + bc1 + bc2,
                            jnp.zeros((2 * dout,), jnp.float32)])
    hcat = _matmul_bias(h, wcat, bcat, bn=1000)
    lin = hcat[:, :dout]
    m1 = hcat[:, dout:2 * dout]
    m2 = hcat[:, 2 * dout:]
    out_a, out_b = _sc_aggregate(
        lin[:, :half], lin[:, half:], m1[:, :half], m1[:, half:],
        m2[:, :half], m2[:, half:], edges1[0], edges1[1],
        edges2[0], edges2[1], half=half)
    return jnp.concatenate([out_a, out_b], axis=1)


def kernel(x, edge_index, edge_weight, edge_index2, edge_weight2,
           Wl1, Wc11, Wc21, bl1, bc11, bc21,
           Wl2, Wc12, Wc22, bl2, bc12, bc22,
           Wl3, Wc13, Wc23, bl3, bc13, bc23):
    edges1 = _pack_edges(edge_index, edge_weight)
    edges2 = _pack_edges(edge_index2, edge_weight2)

    h = _layer(x, Wl1, Wc11, Wc21, bl1, bc11, bc21, edges1, edges2)
    h = _layer(h, Wl2, Wc12, Wc22, bl2, bc12, bc22, edges1, edges2)
    h = _layer(h, Wl3, Wc13, Wc23, bl3, bc13, bc23, edges1, edges2)
    return _log_softmax(h, bn=1000)


# speedup vs baseline: 1.6230x; 1.0160x over previous
import functools

import jax
import jax.numpy as jnp
from jax import lax
from jax.experimental import pallas as pl
from jax.experimental.pallas import tpu as pltpu
from jax.experimental.pallas import tpu_sc as plsc

N = 10000
E = 160000
NSUB = 16
EDGE_B = 112
NB = 90
EP = NSUB * NB * EDGE_B


def _matmul_bias(x, w, b, bn):
    n, k = x.shape
    m = w.shape[1]

    def mm(x_ref, w_ref, b_ref, o_ref):
        o_ref[...] = jnp.dot(x_ref[...], w_ref[...],
                             preferred_element_type=jnp.float32) + b_ref[...]

    return pl.pallas_call(
        mm,
        grid=(n // bn,),
        in_specs=[
            pl.BlockSpec((bn, k), lambda i: (i, 0)),
            pl.BlockSpec((k, m), lambda i: (0, 0)),
            pl.BlockSpec((1, m), lambda i: (0, 0)),
        ],
        out_specs=pl.BlockSpec((bn, m), lambda i: (i, 0)),
        out_shape=jax.ShapeDtypeStruct((n, m), jnp.float32),
    )(x, w, b[None, :])


def _log_softmax(h, bn):
    n, c = h.shape

    def k(h_ref, o_ref):
        v = h_ref[...]
        mx = jnp.max(v, axis=1, keepdims=True)
        e = jnp.exp(v - mx)
        o_ref[...] = v - mx - jnp.log(jnp.sum(e, axis=1, keepdims=True))

    return pl.pallas_call(
        k,
        grid=(n // bn,),
        in_specs=[pl.BlockSpec((bn, c), lambda i: (i, 0))],
        out_specs=pl.BlockSpec((bn, c), lambda i: (i, 0)),
        out_shape=jax.ShapeDtypeStruct((n, c), jnp.float32),
    )(h)


@functools.partial(jax.jit, static_argnames=("half",))
def _sc_aggregate(lin_a, lin_b, m1_a, m1_b, m2_a, m2_b, ed1, ed2, half):
    rows_per_tile = 632
    mesh = plsc.VectorSubcoreMesh(core_axis_name="c", subcore_axis_name="s")

    @functools.partial(
        pl.kernel,
        mesh=mesh,
        compiler_params=pltpu.CompilerParams(use_tc_tiling_on_sc=False,
                                             needs_layout_passes=False),
        out_type=(jax.ShapeDtypeStruct((N, half), jnp.float32),
                  jax.ShapeDtypeStruct((N, half), jnp.float32)),
        scratch_types=[
            pltpu.VMEM((3, 3, EDGE_B), jnp.int32),
            pltpu.VMEM((3, EDGE_B), jnp.int32),
            pltpu.VMEM((EDGE_B, half), jnp.float32),
            pltpu.VMEM((EDGE_B, half), jnp.float32),
            pltpu.VMEM((EDGE_B, half), jnp.float32),
            pltpu.VMEM_SHARED((N, half), jnp.float32),
            pltpu.SemaphoreType.DMA,
            pltpu.SemaphoreType.DMA,
            pltpu.SemaphoreType.DMA,
            pltpu.SemaphoreType.DMA,
            pltpu.SemaphoreType.DMA,
            pltpu.SemaphoreType.DMA,
            pltpu.SemaphoreType.DMA,
            pltpu.SemaphoreType.DMA,
            pltpu.SemaphoreType.DMA,
        ],
    )
    def agg(lin_a_h, lin_b_h, m1_a_h, m1_b_h, m2_a_h, m2_b_h,
            ed1_h, ed2_h, out_a_h, out_b_h,
            ib, sd, rows0, rows1, rows2, acc,
            si0, si1, si2, sg0, sg1, sg2, ss0, ss1, ss2):
        cid = lax.axis_index("c")
        sid = lax.axis_index("s")
        r0 = pl.multiple_of(
            jnp.minimum(sid * rows_per_tile, N - rows_per_tile), 8)
        rows = (rows0, rows1, rows2)
        sem_i = (si0, si1, si2)
        sem_g = (sg0, sg1, sg2)
        sem_s = (ss0, ss1, ss2)

        def one_core(lin_h, m1_h, m2_h, out_h):
            pltpu.sync_copy(lin_h.at[pl.ds(r0, rows_per_tile)],
                            acc.at[pl.ds(r0, rows_per_tile)])
            plsc.subcore_barrier()

            def one_conv(m_h, ed_h):
                def load_idx(b, p):
                    pltpu.async_copy(ed_h.at[sid, b], ib.at[p], sem_i[p])

                def wait_idx(p):
                    pltpu.make_async_copy(ed_h.at[sid, 0], ib.at[p],
                                          sem_i[p]).wait()

                def start_gather(p):
                    pltpu.async_copy(m_h.at[ib.at[p, 0]], rows[p], sem_g[p])

                def wait_gather(p):
                    pltpu.make_async_copy(m_h.at[ib.at[p, 0]],
                                          rows[p], sem_g[p]).wait()

                def start_scatter(p):
                    pltpu.async_copy(rows[p], acc.at[sd.at[p]],
                                     sem_s[p], add=True)

                def wait_scatter(p):
                    pltpu.make_async_copy(rows[p], acc.at[sd.at[p]],
                                          sem_s[p]).wait()

                def scale(p):
                    rbuf = rows[p]

                    def scale_group(g, c2):
                        goff = pl.multiple_of(g * 16, 16)
                        sl = pl.ds(goff, 16)
                        sd[p, sl] = ib[p, 1, sl]
                        w16 = plsc.bitcast(ib[p, 2, sl], jnp.float32)
                        for t in range(16):
                            wbc = w16.at[jnp.full((16,), t, jnp.int32)].get(
                                mode="promise_in_bounds")
                            for q in range(half // 16):
                                qsl = pl.ds(q * 16, 16)
                                rbuf[goff + t, qsl] = rbuf[goff + t, qsl] * wbc
                        return c2

                    lax.fori_loop(0, EDGE_B // 16, scale_group, 0)

                for p in range(3):
                    load_idx(p, p)
                wait_idx(0)
                start_gather(0)

                def step(b, p):
                    wait_gather(p)

                    @pl.when(b >= 2)
                    def _():
                        wait_scatter((p + 1) % 3)

                    @pl.when(b + 1 < NB)
                    def _():
                        wait_idx((p + 1) % 3)
                        start_gather((p + 1) % 3)

                    scale(p)
                    start_scatter(p)

                    @pl.when(b + 3 < NB)
                    def _():
                        load_idx(b + 3, p)

                def trio(k3, c):
                    for i in range(3):
                        step(3 * k3 + i, i)
                    return c

                lax.fori_loop(0, NB // 3, trio, 0)
                wait_scatter((NB - 2) % 3)
                wait_scatter((NB - 1) % 3)

            one_conv(m1_h, ed1_h)
            one_conv(m2_h, ed2_h)
            plsc.subcore_barrier()
            pltpu.sync_copy(acc.at[pl.ds(r0, rows_per_tile)],
                            out_h.at[pl.ds(r0, rows_per_tile)])

        @pl.when(cid == 0)
        def _():
            one_core(lin_a_h, m1_a_h, m2_a_h, out_a_h)

        @pl.when(cid == 1)
        def _():
            one_core(lin_b_h, m1_b_h, m2_b_h, out_b_h)

    return agg(lin_a, lin_b, m1_a, m1_b, m2_a, m2_b, ed1, ed2)


def _pack_edges(edge_index, edge_weight):
    pad = EP - E
    src = jnp.concatenate([edge_index[0], jnp.zeros((pad,), jnp.int32)])
    dst = jnp.concatenate([edge_index[1], jnp.zeros((pad,), jnp.int32)])
    ewb = lax.bitcast_convert_type(
        jnp.concatenate([edge_weight, jnp.zeros((pad,), jnp.float32)]),
        jnp.int32)
    packed = jnp.stack([src, dst, ewb], axis=0).reshape(3, NSUB, NB, EDGE_B)
    return jnp.transpose(packed, (1, 2, 0, 3))


def _layer(h, wl, wc1, wc2, bl, bc1, bc2, edges1, edges2):
    dout = wl.shape[1]
    half = dout // 2
    wcat = jnp.concatenate([wl, wc1, wc2], axis=1)
    bcat = jnp.concatenate([bl + bc1 + bc2,
                            jnp.zeros((2 * dout,), jnp.float32)])
    hcat = _matmul_bias(h, wcat, bcat, bn=1000)
    lin = hcat[:, :dout]
    m1 = hcat[:, dout:2 * dout]
    m2 = hcat[:, 2 * dout:]
    out_a, out_b = _sc_aggregate(
        lin[:, :half], lin[:, half:], m1[:, :half], m1[:, half:],
        m2[:, :half], m2[:, half:], edges1, edges2, half=half)
    return jnp.concatenate([out_a, out_b], axis=1)


def kernel(x, edge_index, edge_weight, edge_index2, edge_weight2,
           Wl1, Wc11, Wc21, bl1, bc11, bc21,
           Wl2, Wc12, Wc22, bl2, bc12, bc22,
           Wl3, Wc13, Wc23, bl3, bc13, bc23):
    edges1 = _pack_edges(edge_index, edge_weight)
    edges2 = _pack_edges(edge_index2, edge_weight2)

    h = _layer(x, Wl1, Wc11, Wc21, bl1, bc11, bc21, edges1, edges2)
    h = _layer(h, Wl2, Wc12, Wc22, bl2, bc12, bc22, edges1, edges2)
    h = _layer(h, Wl3, Wc13, Wc23, bl3, bc13, bc23, edges1, edges2)
    return _log_softmax(h, bn=1000)
